# Initial kernel scaffold; baseline (speedup 1.0000x reference)
import jax
import jax.numpy as jnp
from jax import lax
from jax.experimental import pallas as pl
from jax.experimental.pallas import tpu as pltpu

K = 20
N = 1024
NEG = -3.0e38


def _leaky(z):
    return jnp.where(z > 0, z, 0.2 * z)


def _dgcnn_body(x_ref,
                ec1_W, ec1_b, ec2_W, ec2_b, ec3_W, ec3_b, ec4_W, ec4_b,
                agg_W, agg_b,
                xfeat_ref,
                D_ref, F_ref):
    X = x_ref[0]
    col = 0
    for (C, Cout, li) in ((8, 64, 0), (64, 64, 1), (64, 128, 2), (128, 256, 3)):
        W = (ec1_W, ec2_W, ec3_W, ec4_W)[li][...]
        bvec = (ec1_b, ec2_b, ec3_b, ec4_b)[li][...]
        W1 = W[:C]
        W2 = W[C:]
        A = jnp.dot(X, W1, preferred_element_type=jnp.float32)
        Bv = jnp.dot(X, W2 - W1, preferred_element_type=jnp.float32) + bvec
        sq = jnp.sum(X * X, axis=1)
        G = lax.dot_general(X, X, (((1,), (1,)), ((), ())),
                            preferred_element_type=jnp.float32)
        D_ref[...] = 2.0 * G - sq[:, None] - sq[None, :]

        jj = lax.broadcasted_iota(jnp.int32, (N, N), 1)

        def body(t, M):
            D = D_ref[...]
            m = jnp.max(D, axis=1)
            eq = D == m[:, None]
            idx = jnp.min(jnp.where(eq, jj, N + 1), axis=1)
            H = jj == idx[:, None]
            D_ref[...] = jnp.where(H, NEG, D)
            R = jnp.dot(H.astype(jnp.float32), A,
                        preferred_element_type=jnp.float32)
            return jnp.maximum(M, R)

        M = lax.fori_loop(0, K, body, jnp.full((N, Cout), NEG, jnp.float32))
        Xn = _leaky(M + Bv)
        F_ref[:, col:col + Cout] = Xn
        col += Cout
        X = Xn

    f = _leaky(jnp.dot(F_ref[...], agg_W[...],
                       preferred_element_type=jnp.float32) + agg_b[...])
    xfeat_ref[0] = jnp.max(f, axis=0)


def _head_body(xfeat_ref, metrics_ref,
               tab_W0, tab_b0,
               att_W0, att_b0, feat_W0, feat_b0,
               att_W1, att_b1, feat_W1, feat_b1,
               att_W2, att_b2, feat_W2, feat_b2,
               att_W3, att_b3, feat_W3, feat_b3,
               att_W4, att_b4, feat_W4, feat_b4,
               h1_W, h1_b, h2_W, h2_b, h3_W, h3_b, h4_W, h4_b,
               amax_ref, probs_ref, logits_ref, mfeats_ref):
    metrics = metrics_ref[...]
    n_d = 64
    gamma = 1.5

    def mm(a, w, b):
        return jnp.dot(a, w[...], preferred_element_type=jnp.float32) + b[...]

    prior = jnp.ones_like(metrics)
    feat0 = jnp.maximum(mm(metrics, tab_W0, tab_b0), 0.0)
    a = feat0[:, n_d:]
    atts = ((att_W0, att_b0, feat_W0, feat_b0),
            (att_W1, att_b1, feat_W1, feat_b1),
            (att_W2, att_b2, feat_W2, feat_b2),
            (att_W3, att_b3, feat_W3, feat_b3),
            (att_W4, att_b4, feat_W4, feat_b4))
    for i, (aW, ab, fW, fb) in enumerate(atts):
        mask_logits = mm(a, aW, ab) * prior
        z = mask_logits - jnp.max(mask_logits, axis=-1, keepdims=True)
        ez = jnp.exp(z)
        mask = ez / jnp.sum(ez, axis=-1, keepdims=True)
        prior = prior * (gamma - mask)
        masked = metrics * mask
        feat = jnp.maximum(mm(masked, fW, fb), 0.0)
        mfeats_ref[:, i * n_d:(i + 1) * n_d] = feat[:, :n_d]
        a = feat[:, n_d:]

    feats = jnp.concatenate([xfeat_ref[...], mfeats_ref[...]], axis=1)
    h = jnp.maximum(mm(feats, h1_W, h1_b), 0.0)
    h = jnp.maximum(mm(h, h2_W, h2_b), 0.0)
    h = jnp.maximum(mm(h, h3_W, h3_b), 0.0)
    logits = mm(h, h4_W, h4_b)
    logits_ref[...] = logits
    z = logits - jnp.max(logits, axis=-1, keepdims=True)
    ez = jnp.exp(z)
    probs = ez / jnp.sum(ez, axis=-1, keepdims=True)
    probs_ref[...] = probs
    nc = probs.shape[1]
    ii = lax.broadcasted_iota(jnp.int32, probs.shape, 1)
    pm = jnp.max(probs, axis=1, keepdims=True)
    amax_ref[...] = jnp.min(jnp.where(probs == pm, ii, nc + 1), axis=1,
                            keepdims=True)


def kernel(x, metrics, params):
    p = params
    B = x.shape[0]

    def full(s):
        return pl.BlockSpec(s, lambda *_: (0,) * len(s))

    wspecs = []
    wvals = []
    for name in ('ec1', 'ec2', 'ec3', 'ec4'):
        W = p[name + '_W']
        bv = p[name + '_b'].reshape(1, -1)
        wvals += [W, bv]
        wspecs += [full(W.shape), full(bv.shape)]
    aggW = p['agg_W']
    aggb = p['agg_b'].reshape(1, -1)
    wvals += [aggW, aggb]
    wspecs += [full(aggW.shape), full(aggb.shape)]

    x_feats = pl.pallas_call(
        _dgcnn_body,
        grid=(B,),
        in_specs=[pl.BlockSpec((1, N, x.shape[2]), lambda b: (b, 0, 0))] + wspecs,
        out_specs=pl.BlockSpec((1, 128), lambda b: (b, 0)),
        out_shape=jax.ShapeDtypeStruct((B, 128), jnp.float32),
        scratch_shapes=[
            pltpu.VMEM((N, N), jnp.float32),
            pltpu.VMEM((N, 512), jnp.float32),
        ],
    )(x, *wvals)

    hvals = [x_feats, metrics, p['tab_W0'], p['tab_b0'].reshape(1, -1)]
    for i in range(5):
        hvals += [p['tab_att_W%d' % i], p['tab_att_b%d' % i].reshape(1, -1),
                  p['tab_feat_W%d' % i], p['tab_feat_b%d' % i].reshape(1, -1)]
    for nm in ('h1', 'h2', 'h3', 'h4'):
        hvals += [p[nm + '_W'], p[nm + '_b'].reshape(1, -1)]
    hspecs = [full(v.shape) for v in hvals]

    amax, probs, logits, mfeats = pl.pallas_call(
        _head_body,
        in_specs=hspecs,
        out_specs=[full((B, 1)), full((B, 50)), full((B, 50)), full((B, 320))],
        out_shape=[
            jax.ShapeDtypeStruct((B, 1), jnp.int32),
            jax.ShapeDtypeStruct((B, 50), jnp.float32),
            jax.ShapeDtypeStruct((B, 50), jnp.float32),
            jax.ShapeDtypeStruct((B, 320), jnp.float32),
        ],
    )(*hvals)

    return (amax.reshape(B), probs, logits, x_feats, mfeats)


# speedup vs baseline: 4.4532x; 4.4532x over previous
import jax
import jax.numpy as jnp
from jax import lax
from jax.experimental import pallas as pl
from jax.experimental.pallas import tpu as pltpu

K = 20
N = 1024
NEG = -3.0e38


def _leaky(z):
    return jnp.where(z > 0, z, 0.2 * z)


def _dgcnn_body(x_ref,
                ec1_W, ec1_b, ec2_W, ec2_b, ec3_W, ec3_b, ec4_W, ec4_b,
                agg_W, agg_b,
                xfeat_ref,
                D_ref, F_ref):
    X = x_ref[0]
    col = 0
    for (C, Cout, li) in ((8, 64, 0), (64, 64, 1), (64, 128, 2), (128, 256, 3)):
        W = (ec1_W, ec2_W, ec3_W, ec4_W)[li][...]
        bvec = (ec1_b, ec2_b, ec3_b, ec4_b)[li][...]
        sq = jnp.sum(X * X, axis=1)
        G = lax.dot_general(X, X, (((1,), (1,)), ((), ())),
                            preferred_element_type=jnp.float32)
        D_ref[...] = 2.0 * G - sq[:, None] - sq[None, :]

        jj = lax.broadcasted_iota(jnp.int32, (N, N), 1)

        def body(t, M):
            D = D_ref[...]
            m = jnp.max(D, axis=1)
            eq = D == m[:, None]
            idx = jnp.min(jnp.where(eq, jj, N + 1), axis=1)
            H = jj == idx[:, None]
            D_ref[...] = jnp.where(H, NEG, D)
            S = jnp.dot(H.astype(jnp.float32), X,
                        preferred_element_type=jnp.float32,
                        precision=lax.Precision.HIGHEST)
            e = jnp.concatenate([S - X, X], axis=1)
            h = jnp.dot(e, W, preferred_element_type=jnp.float32) + bvec
            return jnp.maximum(M, h)

        M = lax.fori_loop(0, K, body, jnp.full((N, Cout), NEG, jnp.float32))
        Xn = _leaky(M)
        F_ref[:, col:col + Cout] = Xn
        col += Cout
        X = Xn

    f = _leaky(jnp.dot(F_ref[...], agg_W[...],
                       preferred_element_type=jnp.float32) + agg_b[...])
    xfeat_ref[0, 0] = jnp.max(f, axis=0)


def _head_body(xfeat_ref, metrics_ref,
               tab_W0, tab_b0,
               att_W0, att_b0, feat_W0, feat_b0,
               att_W1, att_b1, feat_W1, feat_b1,
               att_W2, att_b2, feat_W2, feat_b2,
               att_W3, att_b3, feat_W3, feat_b3,
               att_W4, att_b4, feat_W4, feat_b4,
               h1_W, h1_b, h2_W, h2_b, h3_W, h3_b, h4_W, h4_b,
               amax_ref, probs_ref, logits_ref, mfeats_ref):
    metrics = metrics_ref[...]
    n_d = 64
    gamma = 1.5

    def mm(a, w, b):
        return jnp.dot(a, w[...], preferred_element_type=jnp.float32) + b[...]

    prior = jnp.ones_like(metrics)
    feat0 = jnp.maximum(mm(metrics, tab_W0, tab_b0), 0.0)
    a = feat0[:, n_d:]
    atts = ((att_W0, att_b0, feat_W0, feat_b0),
            (att_W1, att_b1, feat_W1, feat_b1),
            (att_W2, att_b2, feat_W2, feat_b2),
            (att_W3, att_b3, feat_W3, feat_b3),
            (att_W4, att_b4, feat_W4, feat_b4))
    for i, (aW, ab, fW, fb) in enumerate(atts):
        mask_logits = mm(a, aW, ab) * prior
        z = mask_logits - jnp.max(mask_logits, axis=-1, keepdims=True)
        ez = jnp.exp(z)
        mask = ez / jnp.sum(ez, axis=-1, keepdims=True)
        prior = prior * (gamma - mask)
        masked = metrics * mask
        feat = jnp.maximum(mm(masked, fW, fb), 0.0)
        mfeats_ref[:, i * n_d:(i + 1) * n_d] = feat[:, :n_d]
        a = feat[:, n_d:]

    feats = jnp.concatenate([xfeat_ref[...], mfeats_ref[...]], axis=1)
    h = jnp.maximum(mm(feats, h1_W, h1_b), 0.0)
    h = jnp.maximum(mm(h, h2_W, h2_b), 0.0)
    h = jnp.maximum(mm(h, h3_W, h3_b), 0.0)
    logits = mm(h, h4_W, h4_b)
    logits_ref[...] = logits
    z = logits - jnp.max(logits, axis=-1, keepdims=True)
    ez = jnp.exp(z)
    probs = ez / jnp.sum(ez, axis=-1, keepdims=True)
    probs_ref[...] = probs
    nc = probs.shape[1]
    ii = lax.broadcasted_iota(jnp.int32, probs.shape, 1)
    pm = jnp.max(probs, axis=1, keepdims=True)
    amax_ref[...] = jnp.min(jnp.where(probs == pm, ii, nc + 1), axis=1,
                            keepdims=True)


def kernel(x, metrics, params):
    p = params
    B = x.shape[0]

    def full(s):
        return pl.BlockSpec(s, lambda *_: (0,) * len(s))

    wspecs = []
    wvals = []
    for name in ('ec1', 'ec2', 'ec3', 'ec4'):
        W = p[name + '_W']
        bv = p[name + '_b'].reshape(1, -1)
        wvals += [W, bv]
        wspecs += [full(W.shape), full(bv.shape)]
    aggW = p['agg_W']
    aggb = p['agg_b'].reshape(1, -1)
    wvals += [aggW, aggb]
    wspecs += [full(aggW.shape), full(aggb.shape)]

    x_feats = pl.pallas_call(
        _dgcnn_body,
        grid=(B,),
        in_specs=[pl.BlockSpec((1, N, x.shape[2]), lambda b: (b, 0, 0))] + wspecs,
        out_specs=pl.BlockSpec((1, 1, 128), lambda b: (b, 0, 0)),
        out_shape=jax.ShapeDtypeStruct((B, 1, 128), jnp.float32),
        scratch_shapes=[
            pltpu.VMEM((N, N), jnp.float32),
            pltpu.VMEM((N, 512), jnp.float32),
        ],
    )(x, *wvals)
    x_feats = x_feats.reshape(B, 128)

    hvals = [x_feats, metrics, p['tab_W0'], p['tab_b0'].reshape(1, -1)]
    for i in range(5):
        hvals += [p['tab_att_W%d' % i], p['tab_att_b%d' % i].reshape(1, -1),
                  p['tab_feat_W%d' % i], p['tab_feat_b%d' % i].reshape(1, -1)]
    for nm in ('h1', 'h2', 'h3', 'h4'):
        hvals += [p[nm + '_W'], p[nm + '_b'].reshape(1, -1)]
    hspecs = [full(v.shape) for v in hvals]

    amax, probs, logits, mfeats = pl.pallas_call(
        _head_body,
        in_specs=hspecs,
        out_specs=[full((B, 1)), full((B, 50)), full((B, 50)), full((B, 320))],
        out_shape=[
            jax.ShapeDtypeStruct((B, 1), jnp.int32),
            jax.ShapeDtypeStruct((B, 50), jnp.float32),
            jax.ShapeDtypeStruct((B, 50), jnp.float32),
            jax.ShapeDtypeStruct((B, 320), jnp.float32),
        ],
    )(*hvals)

    return (amax.reshape(B), probs, logits, x_feats, mfeats)


# speedup vs baseline: 7.7674x; 1.7442x over previous
import jax
import jax.numpy as jnp
from jax import lax
from jax.experimental import pallas as pl
from jax.experimental.pallas import tpu as pltpu

K = 20
N = 1024
NEG = -3.0e38


def _leaky(z):
    return jnp.where(z > 0, z, 0.2 * z)


def _dgcnn_body(x_ref,
                ec1_W, ec1_b, ec2_W, ec2_b, ec3_W, ec3_b, ec4_W, ec4_b,
                agg_W, agg_b,
                xfeat_ref,
                D_ref, F_ref):
    X = x_ref[0]
    col = 0
    for (C, Cout, li) in ((8, 64, 0), (64, 64, 1), (64, 128, 2), (128, 256, 3)):
        W = (ec1_W, ec2_W, ec3_W, ec4_W)[li][...]
        bvec = (ec1_b, ec2_b, ec3_b, ec4_b)[li][...]
        Xhi = X.astype(jnp.bfloat16).astype(jnp.float32)
        r1 = X - Xhi
        Xmid = r1.astype(jnp.bfloat16).astype(jnp.float32)
        Xlo = r1 - Xmid
        sq = jnp.sum(X * X, axis=1)
        G = lax.dot_general(X, X, (((1,), (1,)), ((), ())),
                            preferred_element_type=jnp.float32)
        D_ref[...] = 2.0 * G - sq[:, None] - sq[None, :]

        jj = lax.broadcasted_iota(jnp.int32, (N, N), 1)

        def body(t, M):
            D = D_ref[...]
            m = jnp.max(D, axis=1)
            eq = D == m[:, None]
            idx = jnp.min(jnp.where(eq, jj, N + 1), axis=1)
            H = jj == idx[:, None]
            D_ref[...] = jnp.where(H, NEG, D)
            Hf = H.astype(jnp.float32)
            S = ((jnp.dot(Hf, Xhi, preferred_element_type=jnp.float32)
                  + jnp.dot(Hf, Xmid, preferred_element_type=jnp.float32))
                 + jnp.dot(Hf, Xlo, preferred_element_type=jnp.float32))
            e = jnp.concatenate([S - X, X], axis=1)
            h = jnp.dot(e, W, preferred_element_type=jnp.float32) + bvec
            return jnp.maximum(M, h)

        M = lax.fori_loop(0, K, body, jnp.full((N, Cout), NEG, jnp.float32))
        Xn = _leaky(M)
        F_ref[:, col:col + Cout] = Xn
        col += Cout
        X = Xn

    f = _leaky(jnp.dot(F_ref[...], agg_W[...],
                       preferred_element_type=jnp.float32) + agg_b[...])
    xfeat_ref[0, 0] = jnp.max(f, axis=0)


def _head_body(xfeat_ref, metrics_ref,
               tab_W0, tab_b0,
               att_W0, att_b0, feat_W0, feat_b0,
               att_W1, att_b1, feat_W1, feat_b1,
               att_W2, att_b2, feat_W2, feat_b2,
               att_W3, att_b3, feat_W3, feat_b3,
               att_W4, att_b4, feat_W4, feat_b4,
               h1_W, h1_b, h2_W, h2_b, h3_W, h3_b, h4_W, h4_b,
               amax_ref, probs_ref, logits_ref, mfeats_ref):
    metrics = metrics_ref[...]
    n_d = 64
    gamma = 1.5

    def mm(a, w, b):
        return jnp.dot(a, w[...], preferred_element_type=jnp.float32) + b[...]

    prior = jnp.ones_like(metrics)
    feat0 = jnp.maximum(mm(metrics, tab_W0, tab_b0), 0.0)
    a = feat0[:, n_d:]
    atts = ((att_W0, att_b0, feat_W0, feat_b0),
            (att_W1, att_b1, feat_W1, feat_b1),
            (att_W2, att_b2, feat_W2, feat_b2),
            (att_W3, att_b3, feat_W3, feat_b3),
            (att_W4, att_b4, feat_W4, feat_b4))
    for i, (aW, ab, fW, fb) in enumerate(atts):
        mask_logits = mm(a, aW, ab) * prior
        z = mask_logits - jnp.max(mask_logits, axis=-1, keepdims=True)
        ez = jnp.exp(z)
        mask = ez / jnp.sum(ez, axis=-1, keepdims=True)
        prior = prior * (gamma - mask)
        masked = metrics * mask
        feat = jnp.maximum(mm(masked, fW, fb), 0.0)
        mfeats_ref[:, i * n_d:(i + 1) * n_d] = feat[:, :n_d]
        a = feat[:, n_d:]

    feats = jnp.concatenate([xfeat_ref[...], mfeats_ref[...]], axis=1)
    h = jnp.maximum(mm(feats, h1_W, h1_b), 0.0)
    h = jnp.maximum(mm(h, h2_W, h2_b), 0.0)
    h = jnp.maximum(mm(h, h3_W, h3_b), 0.0)
    logits = mm(h, h4_W, h4_b)
    logits_ref[...] = logits
    z = logits - jnp.max(logits, axis=-1, keepdims=True)
    ez = jnp.exp(z)
    probs = ez / jnp.sum(ez, axis=-1, keepdims=True)
    probs_ref[...] = probs
    nc = probs.shape[1]
    ii = lax.broadcasted_iota(jnp.int32, probs.shape, 1)
    pm = jnp.max(probs, axis=1, keepdims=True)
    amax_ref[...] = jnp.min(jnp.where(probs == pm, ii, nc + 1), axis=1,
                            keepdims=True)


def kernel(x, metrics, params):
    p = params
    B = x.shape[0]

    def full(s):
        return pl.BlockSpec(s, lambda *_: (0,) * len(s))

    wspecs = []
    wvals = []
    for name in ('ec1', 'ec2', 'ec3', 'ec4'):
        W = p[name + '_W']
        bv = p[name + '_b'].reshape(1, -1)
        wvals += [W, bv]
        wspecs += [full(W.shape), full(bv.shape)]
    aggW = p['agg_W']
    aggb = p['agg_b'].reshape(1, -1)
    wvals += [aggW, aggb]
    wspecs += [full(aggW.shape), full(aggb.shape)]

    x_feats = pl.pallas_call(
        _dgcnn_body,
        grid=(B,),
        in_specs=[pl.BlockSpec((1, N, x.shape[2]), lambda b: (b, 0, 0))] + wspecs,
        out_specs=pl.BlockSpec((1, 1, 128), lambda b: (b, 0, 0)),
        out_shape=jax.ShapeDtypeStruct((B, 1, 128), jnp.float32),
        scratch_shapes=[
            pltpu.VMEM((N, N), jnp.float32),
            pltpu.VMEM((N, 512), jnp.float32),
        ],
    )(x, *wvals)
    x_feats = x_feats.reshape(B, 128)

    hvals = [x_feats, metrics, p['tab_W0'], p['tab_b0'].reshape(1, -1)]
    for i in range(5):
        hvals += [p['tab_att_W%d' % i], p['tab_att_b%d' % i].reshape(1, -1),
                  p['tab_feat_W%d' % i], p['tab_feat_b%d' % i].reshape(1, -1)]
    for nm in ('h1', 'h2', 'h3', 'h4'):
        hvals += [p[nm + '_W'], p[nm + '_b'].reshape(1, -1)]
    hspecs = [full(v.shape) for v in hvals]

    amax, probs, logits, mfeats = pl.pallas_call(
        _head_body,
        in_specs=hspecs,
        out_specs=[full((B, 1)), full((B, 50)), full((B, 50)), full((B, 320))],
        out_shape=[
            jax.ShapeDtypeStruct((B, 1), jnp.int32),
            jax.ShapeDtypeStruct((B, 50), jnp.float32),
            jax.ShapeDtypeStruct((B, 50), jnp.float32),
            jax.ShapeDtypeStruct((B, 320), jnp.float32),
        ],
    )(*hvals)

    return (amax.reshape(B), probs, logits, x_feats, mfeats)
